# Initial kernel scaffold; baseline (speedup 1.0000x reference)
import jax
import jax.numpy as jnp
from jax.experimental import pallas as pl


def kernel(x, edge_index, W1, b1, W2, b2):
    raise NotImplementedError("write your pallas kernel here")


# speedup vs baseline: 13.1579x; 13.1579x over previous
import functools

import jax
import jax.numpy as jnp
from jax import lax
from jax.experimental import pallas as pl
from jax.experimental.pallas import tpu as pltpu
from jax.experimental.pallas import tpu_sc as plsc

N = 10000
NP = 10240
E = 320000
D = 128

NC = 2
NS = 16
NW = NC * NS
EPW = E // NW
CH = 80
NCHUNK = EPW // CH
RPT = NP // NS
ZR = 128

_mesh = plsc.VectorSubcoreMesh(core_axis_name="c", subcore_axis_name="s")


def _zero_vec(ref, nwords):
    def body(i, carry):
        ref[pl.ds(i * 16, 16)] = jnp.zeros((16,), jnp.float32)
        return carry
    lax.fori_loop(0, nwords // 16, body, 0)


@functools.partial(
    pl.kernel,
    out_type=jax.ShapeDtypeStruct((NC * NP,), jnp.float32),
    mesh=_mesh,
    scratch_types=[
        pltpu.VMEM((CH,), jnp.int32),
        pltpu.VMEM((CH,), jnp.float32),
        pltpu.VMEM((RPT,), jnp.float32),
        pltpu.VMEM_SHARED((NP,), jnp.float32),
    ],
)
def _sc_deg(dst_hbm, out_hbm, idx_v, ones_v, zbuf_v, deg_sh):
    c = lax.axis_index("c")
    s = lax.axis_index("s")
    wid = s * NC + c

    def set_ones(i, carry):
        ones_v[pl.ds(i * 16, 16)] = jnp.ones((16,), jnp.float32)
        return carry
    lax.fori_loop(0, CH // 16, set_ones, 0)
    _zero_vec(zbuf_v, RPT)
    pltpu.sync_copy(zbuf_v, deg_sh.at[pl.ds(s * RPT, RPT)])
    plsc.subcore_barrier()

    base = wid * EPW

    def body(i, carry):
        pltpu.sync_copy(dst_hbm.at[pl.ds(base + i * CH, CH)], idx_v)
        pltpu.sync_copy(ones_v, deg_sh.at[idx_v], add=True)
        return carry
    lax.fori_loop(0, NCHUNK, body, 0)

    plsc.subcore_barrier()
    pltpu.sync_copy(deg_sh.at[pl.ds(s * RPT, RPT)],
                    out_hbm.at[pl.ds(c * NP + s * RPT, RPT)])


@functools.partial(
    pl.kernel,
    out_type=jax.ShapeDtypeStruct((NC * NP, D), jnp.float32),
    mesh=_mesh,
    scratch_types=[
        pltpu.VMEM((CH,), jnp.int32),
        pltpu.VMEM((CH,), jnp.int32),
        pltpu.VMEM((CH, D), jnp.float32),
        pltpu.VMEM((ZR, D), jnp.float32),
        pltpu.VMEM_SHARED((NP, D), jnp.float32),
        pltpu.SemaphoreType.DMA,
    ],
)
def _sc_agg(src_hbm, dst_hbm, g_hbm, out_hbm,
            src_v, dst_v, rows_v, zbuf_v, acc_sh, sem):
    c = lax.axis_index("c")
    s = lax.axis_index("s")
    wid = s * NC + c

    def zrow(i, carry):
        def zcol(j, carry2):
            zbuf_v[i, pl.ds(j * 16, 16)] = jnp.zeros((16,), jnp.float32)
            return carry2
        lax.fori_loop(0, D // 16, zcol, 0)
        return carry
    lax.fori_loop(0, ZR, zrow, 0)

    def zcopy(k, carry):
        pltpu.sync_copy(zbuf_v, acc_sh.at[pl.ds(s * RPT + k * ZR, ZR)])
        return carry
    lax.fori_loop(0, RPT // ZR, zcopy, 0)
    plsc.subcore_barrier()

    base = wid * EPW

    def body(i, carry):
        off = base + i * CH
        pltpu.sync_copy(src_hbm.at[pl.ds(off, CH)], src_v)
        pltpu.sync_copy(dst_hbm.at[pl.ds(off, CH)], dst_v)
        pltpu.async_copy(g_hbm.at[src_v], rows_v, sem).wait()
        pltpu.sync_copy(rows_v, acc_sh.at[dst_v], add=True)
        return carry
    lax.fori_loop(0, NCHUNK, body, 0)

    plsc.subcore_barrier()

    def dump(k, carry):
        r = s * RPT + k * ZR
        pltpu.sync_copy(acc_sh.at[pl.ds(r, ZR)],
                        out_hbm.at[pl.ds(c * NP + r, ZR)])
        return carry
    lax.fori_loop(0, RPT // ZR, dump, 0)


def _tc_mm_body(x_ref, w_ref, h_ref):
    h_ref[...] = jnp.dot(x_ref[...], w_ref[...],
                         preferred_element_type=jnp.float32)


def _tc_dinv_g_body(degp_ref, h_ref, dinv_ref, g_ref):
    deg = degp_ref[0] + degp_ref[1] + 1.0
    dinv = lax.rsqrt(deg)
    dinv_ref[...] = dinv
    g_ref[...] = dinv * h_ref[...]


def _tc_mid_body(acc_ref, g_ref, dinv_ref, b_ref, w_ref, g2_ref):
    t = dinv_ref[...] * (acc_ref[0] + acc_ref[1] + g_ref[...]) + b_ref[...]
    h = jnp.maximum(t, 0.0)
    g2_ref[...] = dinv_ref[...] * jnp.dot(h, w_ref[...],
                                          preferred_element_type=jnp.float32)


def _tc_fin_body(acc_ref, g_ref, dinv_ref, b_ref, z_ref):
    z_ref[...] = (dinv_ref[...] * (acc_ref[0] + acc_ref[1] + g_ref[...])
                  + b_ref[...])


_tc_mm = pl.pallas_call(
    _tc_mm_body, out_shape=jax.ShapeDtypeStruct((NP, D), jnp.float32))
_tc_dinv_g = pl.pallas_call(
    _tc_dinv_g_body,
    out_shape=(jax.ShapeDtypeStruct((NP, 1), jnp.float32),
               jax.ShapeDtypeStruct((NP, D), jnp.float32)))
_tc_mid = pl.pallas_call(
    _tc_mid_body, out_shape=jax.ShapeDtypeStruct((NP, D), jnp.float32))
_tc_fin = pl.pallas_call(
    _tc_fin_body, out_shape=jax.ShapeDtypeStruct((NP, D), jnp.float32))


@jax.jit
def kernel(x, edge_index, W1, b1, W2, b2):
    src = edge_index[0]
    dst = edge_index[1]
    x_pad = jnp.pad(x, ((0, NP - N), (0, 0)))
    b1r = b1.reshape(1, D)
    b2r = b2.reshape(1, D)

    degp = _sc_deg(dst).reshape(NC, NP, 1)
    h1 = _tc_mm(x_pad, W1)
    dinv, g1 = _tc_dinv_g(degp, h1)
    acc1 = _sc_agg(src, dst, g1).reshape(NC, NP, D)
    g2 = _tc_mid(acc1, g1, dinv, b1r, W2)
    acc2 = _sc_agg(src, dst, g2).reshape(NC, NP, D)
    z = _tc_fin(acc2, g2, dinv, b2r)
    return z[:N]


# speedup vs baseline: 16.5693x; 1.2593x over previous
import functools

import jax
import jax.numpy as jnp
from jax import lax
from jax.experimental import pallas as pl
from jax.experimental.pallas import tpu as pltpu
from jax.experimental.pallas import tpu_sc as plsc

N = 10000
NP = 10240
E = 320000
D = 128

NC = 2
NS = 16
NW = NC * NS
EPW = E // NW
RPT = NP // NS

DCH = 80
DNCH = EPW // DCH

CH = 120
NCH = 84
EPT = NCH * CH
E2 = NW * EPT
NSL = 3
ZCP = 80

_mesh = plsc.VectorSubcoreMesh(core_axis_name="c", subcore_axis_name="s")


@functools.partial(
    pl.kernel,
    out_type=jax.ShapeDtypeStruct((NC * NP,), jnp.float32),
    mesh=_mesh,
    scratch_types=[
        pltpu.VMEM((DCH,), jnp.int32),
        pltpu.VMEM((DCH,), jnp.float32),
        pltpu.VMEM((640,), jnp.float32),
        pltpu.VMEM_SHARED((NP,), jnp.float32),
    ],
)
def _sc_deg(dst_hbm, out_hbm, idx_v, ones_v, zbuf_v, deg_sh):
    c = lax.axis_index("c")
    s = lax.axis_index("s")
    wid = s * NC + c

    def set_ones(i, carry):
        ones_v[pl.ds(i * 16, 16)] = jnp.ones((16,), jnp.float32)
        return carry
    lax.fori_loop(0, DCH // 16, set_ones, 0)

    def set_zero(i, carry):
        zbuf_v[pl.ds(i * 16, 16)] = jnp.zeros((16,), jnp.float32)
        return carry
    lax.fori_loop(0, 640 // 16, set_zero, 0)
    pltpu.sync_copy(zbuf_v.at[pl.ds(0, RPT)], deg_sh.at[pl.ds(s * RPT, RPT)])
    plsc.subcore_barrier()

    base = wid * EPW

    def body(i, carry):
        pltpu.sync_copy(dst_hbm.at[pl.ds(base + i * DCH, DCH)], idx_v)
        pltpu.sync_copy(ones_v, deg_sh.at[idx_v], add=True)
        return carry
    lax.fori_loop(0, DNCH, body, 0)

    plsc.subcore_barrier()
    pltpu.sync_copy(deg_sh.at[pl.ds(s * RPT, RPT)],
                    out_hbm.at[pl.ds(c * NP + s * RPT, RPT)])


@functools.partial(
    pl.kernel,
    out_type=jax.ShapeDtypeStruct((NC * NP, D), jnp.float32),
    mesh=_mesh,
    scratch_types=[
        [pltpu.VMEM((CH,), jnp.int32) for _ in range(NSL)],
        [pltpu.VMEM((CH,), jnp.int32) for _ in range(NSL)],
        [pltpu.VMEM((CH, D), jnp.float32) for _ in range(NSL)],
        pltpu.VMEM_SHARED((NP, D), jnp.float32),
        [pltpu.SemaphoreType.DMA for _ in range(NSL)],
        [pltpu.SemaphoreType.DMA for _ in range(NSL)],
        [pltpu.SemaphoreType.DMA for _ in range(NSL)],
    ],
)
def _sc_agg(src_hbm, dst_hbm, g_hbm, out_hbm,
            src_v, dst_v, rows, acc_sh, sem_i, sem_g, sem_s):
    c = lax.axis_index("c")
    s = lax.axis_index("s")
    wid = s * NC + c

    def zrow(i, carry):
        def zcol(j, carry2):
            rows[0][i, pl.ds(j * 16, 16)] = jnp.zeros((16,), jnp.float32)
            return carry2
        lax.fori_loop(0, D // 16, zcol, 0)
        return carry
    lax.fori_loop(0, ZCP, zrow, 0)

    def zcopy(k, carry):
        pltpu.sync_copy(rows[0].at[pl.ds(0, ZCP)],
                        acc_sh.at[pl.ds(s * RPT + k * ZCP, ZCP)])
        return carry
    lax.fori_loop(0, RPT // ZCP, zcopy, 0)
    plsc.subcore_barrier()

    def idx_copies(g, j):
        return (pltpu.make_async_copy(src_hbm.at[wid, g], src_v[j], sem_i[j]),
                pltpu.make_async_copy(dst_hbm.at[wid, g], dst_v[j], sem_i[j]))

    def gather_copy(j):
        return pltpu.make_async_copy(g_hbm.at[src_v[j]], rows[j], sem_g[j])

    def scatter_copy(j):
        return pltpu.make_async_copy(rows[j], acc_sh.at[dst_v[j]], sem_s[j])

    def step(t, carry):
        for r in range(NSL):
            g = 3 * t + r
            j0 = r
            j1 = (r - 1) % NSL
            j2 = (r - 2) % NSL

            @pl.when(g >= 3)
            def _():
                scatter_copy(j0).wait()

            @pl.when(g < NCH)
            def _():
                ic1, ic2 = idx_copies(g, j0)
                ic1.start()
                ic2.start()

            @pl.when(jnp.logical_and(g >= 1, g < NCH + 1))
            def _():
                ic1, ic2 = idx_copies(g - 1, j1)
                ic1.wait()
                ic2.wait()
                gather_copy(j1).start()

            @pl.when(jnp.logical_and(g >= 2, g < NCH + 2))
            def _():
                gather_copy(j2).wait()
                sc = pltpu.async_copy(rows[j2], acc_sh.at[dst_v[j2]],
                                      sem_s[j2], add=True)
        return carry
    lax.fori_loop(0, NCH // 3 + 1, step, 0)

    plsc.subcore_barrier()

    def dump(k, carry):
        r = s * RPT + k * ZCP
        pltpu.sync_copy(acc_sh.at[pl.ds(r, ZCP)],
                        out_hbm.at[pl.ds(c * NP + r, ZCP)])
        return carry
    lax.fori_loop(0, RPT // ZCP, dump, 0)


def _tc_mm_body(x_ref, w_ref, h_ref):
    h_ref[...] = jnp.dot(x_ref[...], w_ref[...],
                         preferred_element_type=jnp.float32)


def _tc_dinv_g_body(degp_ref, h_ref, dinv_ref, g_ref):
    deg = degp_ref[0] + degp_ref[1] + 1.0
    dinv = lax.rsqrt(deg)
    dinv_ref[...] = dinv
    g_ref[...] = dinv * h_ref[...]


def _tc_mid_body(acc_ref, g_ref, dinv_ref, b_ref, w_ref, g2_ref):
    t = dinv_ref[...] * (acc_ref[0] + acc_ref[1] + g_ref[...]) + b_ref[...]
    h = jnp.maximum(t, 0.0)
    g2_ref[...] = dinv_ref[...] * jnp.dot(h, w_ref[...],
                                          preferred_element_type=jnp.float32)


def _tc_fin_body(acc_ref, g_ref, dinv_ref, b_ref, z_ref):
    z_ref[...] = (dinv_ref[...] * (acc_ref[0] + acc_ref[1] + g_ref[...])
                  + b_ref[...])


_tc_mm = pl.pallas_call(
    _tc_mm_body, out_shape=jax.ShapeDtypeStruct((NP, D), jnp.float32))
_tc_dinv_g = pl.pallas_call(
    _tc_dinv_g_body,
    out_shape=(jax.ShapeDtypeStruct((NP, 1), jnp.float32),
               jax.ShapeDtypeStruct((NP, D), jnp.float32)))
_tc_mid = pl.pallas_call(
    _tc_mid_body, out_shape=jax.ShapeDtypeStruct((NP, D), jnp.float32))
_tc_fin = pl.pallas_call(
    _tc_fin_body, out_shape=jax.ShapeDtypeStruct((NP, D), jnp.float32))


@jax.jit
def kernel(x, edge_index, W1, b1, W2, b2):
    src = edge_index[0]
    dst = edge_index[1]
    x_pad = jnp.pad(x, ((0, NP - N), (0, 0)))
    b1r = b1.reshape(1, D)
    b2r = b2.reshape(1, D)
    pad = jnp.full((E2 - E,), N, dtype=src.dtype)
    srcp = jnp.concatenate([src, pad]).reshape(NW, NCH, CH)
    dstp = jnp.concatenate([dst, pad]).reshape(NW, NCH, CH)

    degp = _sc_deg(dst).reshape(NC, NP, 1)
    h1 = _tc_mm(x_pad, W1)
    dinv, g1 = _tc_dinv_g(degp, h1)
    acc1 = _sc_agg(srcp, dstp, g1).reshape(NC, NP, D)
    g2 = _tc_mid(acc1, g1, dinv, b1r, W2)
    acc2 = _sc_agg(srcp, dstp, g2).reshape(NC, NP, D)
    z = _tc_fin(acc2, g2, dinv, b2r)
    return z[:N]


# speedup vs baseline: 25.3210x; 1.5282x over previous
import functools

import jax
import jax.numpy as jnp
from jax import lax
from jax.experimental import pallas as pl
from jax.experimental.pallas import tpu as pltpu
from jax.experimental.pallas import tpu_sc as plsc

N = 10000
NP = 10240
E = 320000
D = 128

NC = 2
NS = 16
RPT = NP // NS

CH = 120
CPP = 167
NCH0 = 114
NCH1 = CPP - NCH0
TOTCH = NS * CPP
E2 = TOTCH * CH
NSL = 3
NT = (NCH0 + 3 + 2) // 3
ZCP = 80

_mesh = plsc.VectorSubcoreMesh(core_axis_name="c", subcore_axis_name="s")


@functools.partial(
    pl.kernel,
    out_type=jax.ShapeDtypeStruct((NC * NP,), jnp.float32),
    mesh=_mesh,
    scratch_types=[
        [pltpu.VMEM((CH,), jnp.int32) for _ in range(NSL)],
        pltpu.VMEM((CH,), jnp.float32),
        pltpu.VMEM((640,), jnp.float32),
        pltpu.VMEM_SHARED((NP,), jnp.float32),
        [pltpu.SemaphoreType.DMA for _ in range(NSL)],
        [pltpu.SemaphoreType.DMA for _ in range(NSL)],
    ],
)
def _sc_deg(dst_hbm, out_hbm, idx_v, ones_v, zbuf_v, deg_sh, sem_i, sem_d):
    c = lax.axis_index("c")
    s = lax.axis_index("s")
    base = s * CPP + c * NCH0
    ncht = jnp.where(c == 0, NCH0, NCH1)

    def set_ones(i, carry):
        ones_v[pl.ds(i * 16, 16)] = jnp.ones((16,), jnp.float32)
        return carry
    lax.fori_loop(0, CH // 16, set_ones, 0)

    def set_zero(i, carry):
        zbuf_v[pl.ds(i * 16, 16)] = jnp.zeros((16,), jnp.float32)
        return carry
    lax.fori_loop(0, 640 // 16, set_zero, 0)
    pltpu.sync_copy(zbuf_v.at[pl.ds(0, RPT)], deg_sh.at[pl.ds(s * RPT, RPT)])
    plsc.subcore_barrier()

    def idx_copy(g, j):
        return pltpu.make_async_copy(dst_hbm.at[base + g], idx_v[j], sem_i[j])

    def scat_copy(j):
        return pltpu.make_async_copy(ones_v, deg_sh.at[idx_v[j]], sem_d[j])

    def step(t, carry):
        for r in range(NSL):
            g = 3 * t + r
            j0 = r
            j1 = (r - 1) % NSL
            j2 = (r - 2) % NSL

            @pl.when(jnp.logical_and(g >= 2, g < ncht + 2))
            def _():
                scat_copy(j2).wait()

            @pl.when(g < ncht)
            def _():
                idx_copy(g, j0).start()

            @pl.when(jnp.logical_and(g >= 1, g < ncht + 1))
            def _():
                idx_copy(g - 1, j1).wait()
                pltpu.async_copy(ones_v, deg_sh.at[idx_v[j1]], sem_d[j1],
                                 add=True)
        return carry
    lax.fori_loop(0, NT, step, 0)

    plsc.subcore_barrier()
    pltpu.sync_copy(deg_sh.at[pl.ds(s * RPT, RPT)],
                    out_hbm.at[pl.ds(c * NP + s * RPT, RPT)])


@functools.partial(
    pl.kernel,
    out_type=jax.ShapeDtypeStruct((NC * NP, D), jnp.float32),
    mesh=_mesh,
    scratch_types=[
        [pltpu.VMEM((CH,), jnp.int32) for _ in range(NSL)],
        [pltpu.VMEM((CH,), jnp.int32) for _ in range(NSL)],
        [pltpu.VMEM((CH, D), jnp.float32) for _ in range(NSL)],
        pltpu.VMEM_SHARED((NP, D), jnp.float32),
        [pltpu.SemaphoreType.DMA for _ in range(NSL)],
        [pltpu.SemaphoreType.DMA for _ in range(NSL)],
        [pltpu.SemaphoreType.DMA for _ in range(NSL)],
    ],
)
def _sc_agg(src_hbm, dst_hbm, g_hbm, out_hbm,
            src_v, dst_v, rows, acc_sh, sem_i, sem_g, sem_s):
    c = lax.axis_index("c")
    s = lax.axis_index("s")
    base = s * CPP + c * NCH0
    ncht = jnp.where(c == 0, NCH0, NCH1)

    def zrow(i, carry):
        def zcol(j, carry2):
            rows[0][i, pl.ds(j * 16, 16)] = jnp.zeros((16,), jnp.float32)
            return carry2
        lax.fori_loop(0, D // 16, zcol, 0)
        return carry
    lax.fori_loop(0, ZCP, zrow, 0)

    def zcopy(k, carry):
        pltpu.sync_copy(rows[0].at[pl.ds(0, ZCP)],
                        acc_sh.at[pl.ds(s * RPT + k * ZCP, ZCP)])
        return carry
    lax.fori_loop(0, RPT // ZCP, zcopy, 0)
    plsc.subcore_barrier()

    def idx_copies(g, j):
        return (pltpu.make_async_copy(src_hbm.at[base + g], src_v[j],
                                      sem_i[j]),
                pltpu.make_async_copy(dst_hbm.at[base + g], dst_v[j],
                                      sem_i[j]))

    def gather_copy(j):
        return pltpu.make_async_copy(g_hbm.at[src_v[j]], rows[j], sem_g[j])

    def scatter_copy(j):
        return pltpu.make_async_copy(rows[j], acc_sh.at[dst_v[j]], sem_s[j])

    def step(t, carry):
        for r in range(NSL):
            g = 3 * t + r
            j0 = r
            j1 = (r - 1) % NSL
            j2 = (r - 2) % NSL

            @pl.when(jnp.logical_and(g >= 3, g < ncht + 3))
            def _():
                scatter_copy(j0).wait()

            @pl.when(g < ncht)
            def _():
                ic1, ic2 = idx_copies(g, j0)
                ic1.start()
                ic2.start()

            @pl.when(jnp.logical_and(g >= 1, g < ncht + 1))
            def _():
                ic1, ic2 = idx_copies(g - 1, j1)
                ic1.wait()
                ic2.wait()
                gather_copy(j1).start()

            @pl.when(jnp.logical_and(g >= 2, g < ncht + 2))
            def _():
                gather_copy(j2).wait()
                pltpu.async_copy(rows[j2], acc_sh.at[dst_v[j2]],
                                 sem_s[j2], add=True)
        return carry
    lax.fori_loop(0, NT, step, 0)

    plsc.subcore_barrier()

    def dump(k, carry):
        r = s * RPT + k * ZCP
        pltpu.sync_copy(acc_sh.at[pl.ds(r, ZCP)],
                        out_hbm.at[pl.ds(c * NP + r, ZCP)])
        return carry
    lax.fori_loop(0, RPT // ZCP, dump, 0)


def _tc_mm_body(x_ref, w_ref, h_ref):
    h_ref[...] = jnp.dot(x_ref[...], w_ref[...],
                         preferred_element_type=jnp.float32)


def _tc_dinv_g_body(degp_ref, h_ref, dinv_ref, g_ref):
    deg = degp_ref[0] + degp_ref[1] + 1.0
    dinv = lax.rsqrt(deg)
    dinv_ref[...] = dinv
    g_ref[...] = dinv * h_ref[...]


def _tc_mid_body(acc_ref, g_ref, dinv_ref, b_ref, w_ref, g2_ref):
    t = dinv_ref[...] * (acc_ref[0] + acc_ref[1] + g_ref[...]) + b_ref[...]
    h = jnp.maximum(t, 0.0)
    g2_ref[...] = dinv_ref[...] * jnp.dot(h, w_ref[...],
                                          preferred_element_type=jnp.float32)


def _tc_fin_body(acc_ref, g_ref, dinv_ref, b_ref, z_ref):
    z_ref[...] = (dinv_ref[...] * (acc_ref[0] + acc_ref[1] + g_ref[...])
                  + b_ref[...])


_tc_mm = pl.pallas_call(
    _tc_mm_body, out_shape=jax.ShapeDtypeStruct((NP, D), jnp.float32))
_tc_dinv_g = pl.pallas_call(
    _tc_dinv_g_body,
    out_shape=(jax.ShapeDtypeStruct((NP, 1), jnp.float32),
               jax.ShapeDtypeStruct((NP, D), jnp.float32)))
_tc_mid = pl.pallas_call(
    _tc_mid_body, out_shape=jax.ShapeDtypeStruct((NP, D), jnp.float32))
_tc_fin = pl.pallas_call(
    _tc_fin_body, out_shape=jax.ShapeDtypeStruct((NP, D), jnp.float32))


@jax.jit
def kernel(x, edge_index, W1, b1, W2, b2):
    src = edge_index[0]
    dst = edge_index[1]
    x_pad = jnp.pad(x, ((0, NP - N), (0, 0)))
    b1r = b1.reshape(1, D)
    b2r = b2.reshape(1, D)
    pad = jnp.full((E2 - E,), N, dtype=src.dtype)
    srcp = jnp.concatenate([src, pad]).reshape(TOTCH, CH)
    dstp = jnp.concatenate([dst, pad]).reshape(TOTCH, CH)

    degp = _sc_deg(dstp).reshape(NC, NP, 1)
    h1 = _tc_mm(x_pad, W1)
    dinv, g1 = _tc_dinv_g(degp, h1)
    acc1 = _sc_agg(srcp, dstp, g1).reshape(NC, NP, D)
    g2 = _tc_mid(acc1, g1, dinv, b1r, W2)
    acc2 = _sc_agg(srcp, dstp, g2).reshape(NC, NP, D)
    z = _tc_fin(acc2, g2, dinv, b2r)
    return z[:N]
